# baseline (speedup 1.0000x reference)
import functools

import jax
import jax.numpy as jnp
from jax import lax
from jax.experimental import pallas as pl
from jax.experimental.pallas import tpu as pltpu
from jax.experimental.pallas import tpu_sc as plsc

N = 10000
NP = 10112
E = 320000
D = 128
G = 16
NC = 2
NS = 16
NW = NC * NS
EP = 327680
EW = EP // NW
C = 128
NCH = EW // C
PS = NP // NS
ZR = 128


@functools.cache
def _sc_mesh():
    return plsc.VectorSubcoreMesh(
        core_axis_name="c", subcore_axis_name="s",
        num_cores=NC, num_subcores=NS)


def _edge_kernel(ei, hp):
    @functools.partial(
        pl.kernel,
        out_type=jax.ShapeDtypeStruct((NC * NP, D), jnp.float32),
        mesh=_sc_mesh(),
        scratch_types=[
            pltpu.VMEM((C, D), jnp.float32),
            pltpu.VMEM((C, D), jnp.float32),
            pltpu.VMEM((C, D), jnp.float32),
            pltpu.VMEM((2, C), jnp.int32),
            pltpu.VMEM((2, C), jnp.int32),
            pltpu.VMEM((2, C), jnp.int32),
            pltpu.VMEM_SHARED((NP, D), jnp.float32),
            pltpu.SemaphoreType.DMA,
            pltpu.SemaphoreType.DMA,
            pltpu.SemaphoreType.DMA,
            pltpu.SemaphoreType.DMA,
            pltpu.SemaphoreType.DMA,
            pltpu.SemaphoreType.DMA,
            pltpu.SemaphoreType.DMA,
            pltpu.SemaphoreType.DMA,
            pltpu.SemaphoreType.DMA,
        ],
    )
    def k(ei_hbm, hp_hbm, part_hbm, rows0, rows1, rows2, ib0, ib1, ib2,
          accum, isem0, isem1, isem2, gsem0, gsem1, gsem2,
          ssem0, ssem1, ssem2):
        cidx = lax.axis_index("c")
        sidx = lax.axis_index("s")
        base = (cidx * NS + sidx) * EW

        rbase = sidx * PS
        last_s = sidx == NS - 1

        def init_from_hp(nrows):
            for kk in range(nrows // ZR):
                pltpu.sync_copy(hp_hbm.at[pl.ds(rbase + kk * ZR, ZR)], rows0)
                pltpu.sync_copy(rows0, accum.at[pl.ds(rbase + kk * ZR, ZR)])
            rem = nrows - (nrows // ZR) * ZR
            if rem:
                pltpu.sync_copy(
                    hp_hbm.at[pl.ds(rbase + (nrows // ZR) * ZR, rem)],
                    rows0.at[pl.ds(0, rem)])
                pltpu.sync_copy(
                    rows0.at[pl.ds(0, rem)],
                    accum.at[pl.ds(rbase + (nrows // ZR) * ZR, rem)])

        @pl.when(jnp.logical_and(cidx == 0, jnp.logical_not(last_s)))
        def _():
            init_from_hp(PS)

        @pl.when(jnp.logical_and(cidx == 0, last_s))
        def _():
            init_from_hp(N - (NS - 1) * PS)

        @pl.when(cidx != 0)
        def _():
            @pl.loop(0, C)
            def _(r):
                for q in range(D // 16):
                    rows0[r, pl.ds(q * 16, 16)] = jnp.zeros((16,), jnp.float32)
            for kk in range(PS // ZR):
                pltpu.sync_copy(rows0, accum.at[pl.ds(rbase + kk * ZR, ZR)])
            rem = PS - (PS // ZR) * ZR
            pltpu.sync_copy(rows0.at[pl.ds(0, rem)],
                            accum.at[pl.ds(rbase + (PS // ZR) * ZR, rem)])

        plsc.subcore_barrier()

        bufs = ((rows0, ib0, isem0, gsem0, ssem0),
                (rows1, ib1, isem1, gsem1, ssem1),
                (rows2, ib2, isem2, gsem2, ssem2))

        def load_idx(j, ib, sem):
            pltpu.async_copy(ei_hbm.at[:, pl.ds(base + j * C, C)], ib, sem)

        def idx_wait(ib, sem):
            pltpu.make_async_copy(ei_hbm.at[:, pl.ds(0, C)], ib, sem).wait()

        def gath(ib, rows, sem):
            pltpu.async_copy(hp_hbm.at[ib.at[0]], rows, sem)

        def gath_wait(ib, rows, sem):
            pltpu.make_async_copy(hp_hbm.at[ib.at[0]], rows, sem).wait()

        def scat(rows, ib, sem):
            pltpu.async_copy(rows, accum.at[ib.at[1]], sem, add=True)

        def scat_wait(rows, ib, sem):
            pltpu.make_async_copy(rows, accum.at[ib.at[1]], sem).wait()

        for b, (rows, ib, isem, gsem, ssem) in enumerate(bufs):
            load_idx(b, ib, isem)
        for b, (rows, ib, isem, gsem, ssem) in enumerate(bufs):
            idx_wait(ib, isem)
            gath(ib, rows, gsem)

        LI = NCH // 3 - 1

        @pl.loop(0, LI)
        def _(i):
            j0 = i * 3
            for b, (rows, ib, isem, gsem, ssem) in enumerate(bufs):
                gath_wait(ib, rows, gsem)
                scat(rows, ib, ssem)
                scat_wait(rows, ib, ssem)
                load_idx(j0 + b + 3, ib, isem)
                idx_wait(ib, isem)
                gath(ib, rows, gsem)

        for b, (rows, ib, isem, gsem, ssem) in enumerate(bufs):
            gath_wait(ib, rows, gsem)
            scat(rows, ib, ssem)
            scat_wait(rows, ib, ssem)
            if b < 2:
                load_idx(NCH - 2 + b, ib, isem)
                idx_wait(ib, isem)
                gath(ib, rows, gsem)
        for b, (rows, ib, isem, gsem, ssem) in enumerate(bufs[:2]):
            gath_wait(ib, rows, gsem)
            scat(rows, ib, ssem)
            scat_wait(rows, ib, ssem)

        plsc.subcore_barrier()
        pltpu.sync_copy(accum.at[pl.ds(rbase, PS)],
                        part_hbm.at[pl.ds(cidx * NP + rbase, PS)])

    return k(ei, hp)


def _deg_kernel(ei):
    @functools.partial(
        pl.kernel,
        out_type=jax.ShapeDtypeStruct((NC * NP, D), jnp.float32),
        mesh=_sc_mesh(),
        scratch_types=[
            pltpu.VMEM((C, D), jnp.float32),
            pltpu.VMEM((C, D), jnp.float32),
            pltpu.VMEM((2, C), jnp.int32),
            pltpu.VMEM((2, C), jnp.int32),
            pltpu.VMEM((2, C), jnp.int32),
            pltpu.VMEM_SHARED((NP, D), jnp.float32),
            pltpu.SemaphoreType.DMA,
            pltpu.SemaphoreType.DMA,
            pltpu.SemaphoreType.DMA,
            pltpu.SemaphoreType.DMA,
            pltpu.SemaphoreType.DMA,
            pltpu.SemaphoreType.DMA,
        ],
    )
    def k(ei_hbm, deg_hbm, ones_rows, zero_rows, ib0, ib1, ib2,
          accum, isem0, isem1, isem2, ssem0, ssem1, ssem2):
        cidx = lax.axis_index("c")
        sidx = lax.axis_index("s")
        base = (cidx * NS + sidx) * EW
        rbase = sidx * PS

        @pl.loop(0, C)
        def _(r):
            for q in range(D // 16):
                ones_rows[r, pl.ds(q * 16, 16)] = jnp.ones((16,), jnp.float32)
                zero_rows[r, pl.ds(q * 16, 16)] = jnp.zeros((16,), jnp.float32)

        @pl.when(cidx == 0)
        def _():
            for kk in range(PS // ZR):
                pltpu.sync_copy(ones_rows, accum.at[pl.ds(rbase + kk * ZR, ZR)])
            rem = PS - (PS // ZR) * ZR
            pltpu.sync_copy(ones_rows.at[pl.ds(0, rem)],
                            accum.at[pl.ds(rbase + (PS // ZR) * ZR, rem)])

        @pl.when(cidx != 0)
        def _():
            for kk in range(PS // ZR):
                pltpu.sync_copy(zero_rows, accum.at[pl.ds(rbase + kk * ZR, ZR)])
            rem = PS - (PS // ZR) * ZR
            pltpu.sync_copy(zero_rows.at[pl.ds(0, rem)],
                            accum.at[pl.ds(rbase + (PS // ZR) * ZR, rem)])

        plsc.subcore_barrier()

        bufs = ((ib0, isem0, ssem0), (ib1, isem1, ssem1), (ib2, isem2, ssem2))

        def load_idx(j, ib, sem):
            pltpu.async_copy(ei_hbm.at[:, pl.ds(base + j * C, C)], ib, sem)

        def idx_wait(ib, sem):
            pltpu.make_async_copy(ei_hbm.at[:, pl.ds(0, C)], ib, sem).wait()

        def scat(ib, sem):
            pltpu.async_copy(ones_rows, accum.at[ib.at[1]], sem, add=True)

        def scat_wait(ib, sem):
            pltpu.make_async_copy(ones_rows, accum.at[ib.at[1]], sem).wait()

        for b, (ib, isem, ssem) in enumerate(bufs):
            load_idx(b, ib, isem)

        LI = NCH // 3 - 1

        @pl.loop(0, LI)
        def _(i):
            j0 = i * 3
            for b, (ib, isem, ssem) in enumerate(bufs):
                idx_wait(ib, isem)
                scat(ib, ssem)
                scat_wait(ib, ssem)
                load_idx(j0 + b + 3, ib, isem)

        for b, (ib, isem, ssem) in enumerate(bufs):
            idx_wait(ib, isem)
            scat(ib, ssem)
            scat_wait(ib, ssem)
            if b < 2:
                load_idx(NCH - 2 + b, ib, isem)
        for b, (ib, isem, ssem) in enumerate(bufs[:2]):
            idx_wait(ib, isem)
            scat(ib, ssem)
            scat_wait(ib, ssem)

        plsc.subcore_barrier()
        pltpu.sync_copy(accum.at[pl.ds(rbase, PS)],
                        deg_hbm.at[pl.ds(cidx * NP + rbase, PS)])

    return k(ei)


def _tc_first(x, W, dinv_b):
    def body(x_ref, w_ref, d_ref, o_ref):
        h = jnp.dot(x_ref[...], w_ref[...], preferred_element_type=jnp.float32)
        o_ref[...] = h * lax.rsqrt(d_ref[:N, 0:1] + d_ref[NP:NP + N, 0:1])

    return pl.pallas_call(
        body, out_shape=jax.ShapeDtypeStruct((N, D), jnp.float32))(x, W, dinv_b)


def _tc_mid(part, dinv_b, b, W):
    def body(p_ref, d_ref, b_ref, w_ref, o_ref):
        agg = p_ref[:N, :] + p_ref[NP:NP + N, :]
        d = lax.rsqrt(d_ref[:N, 0:1] + d_ref[NP:NP + N, 0:1])
        a = jnp.maximum(agg * d + b_ref[...], 0.0)
        o_ref[...] = jnp.dot(
            a, w_ref[...], preferred_element_type=jnp.float32) * d

    return pl.pallas_call(
        body, out_shape=jax.ShapeDtypeStruct((N, D), jnp.float32))(
            part, dinv_b, b, W)


def _tc_final(part, dinv_b, b, bidx):
    def body(p_ref, d_ref, b_ref, bi_ref, o_ref):
        agg = p_ref[:N, :] + p_ref[NP:NP + N, :]
        d = lax.rsqrt(d_ref[:N, 0:1] + d_ref[NP:NP + N, 0:1])
        a = jnp.maximum(agg * d + b_ref[...], 0.0)
        gids = lax.broadcasted_iota(jnp.int32, (G, N), 0).astype(jnp.float32)
        mask = jnp.where(gids == bi_ref[0:1, :], 1.0, 0.0)
        sums = jnp.dot(mask, a, preferred_element_type=jnp.float32)
        counts = jnp.sum(mask, axis=1, keepdims=True)
        o_ref[...] = sums / jnp.maximum(counts, 1.0)

    return pl.pallas_call(
        body, out_shape=jax.ShapeDtypeStruct((G, D), jnp.float32))(
            part, dinv_b, b, bidx)


def kernel(x, edge_index, batch_idx, W1, b1, W2, b2, W3, b3):
    ei = edge_index.astype(jnp.int32)
    pad = jnp.broadcast_to(
        jnp.array([[0], [N]], jnp.int32), (2, EP - E))
    ei = jnp.concatenate([ei, pad], axis=1)
    bidx = jnp.broadcast_to(batch_idx.astype(jnp.float32)[None, :], (8, N))

    deg_b = _deg_kernel(ei)
    hp = _tc_first(x, W1, deg_b)
    part = _edge_kernel(ei, hp)
    hp = _tc_mid(part, deg_b, b1.reshape(1, D), W2)
    part = _edge_kernel(ei, hp)
    hp = _tc_mid(part, deg_b, b2.reshape(1, D), W3)
    part = _edge_kernel(ei, hp)
    return _tc_final(part, deg_b, b3.reshape(1, D), bidx)


# speedup vs baseline: 1.0055x; 1.0055x over previous
import functools

import jax
import jax.numpy as jnp
from jax import lax
from jax.experimental import pallas as pl
from jax.experimental.pallas import tpu as pltpu
from jax.experimental.pallas import tpu_sc as plsc

N = 10000
NP = 10112
E = 320000
D = 128
G = 16
NC = 2
NS = 16
NW = NC * NS
EP = 327680
EW = EP // NW
C = 128
NCH = EW // C
PS = NP // NS
ZR = 128


@functools.cache
def _sc_mesh():
    return plsc.VectorSubcoreMesh(
        core_axis_name="c", subcore_axis_name="s",
        num_cores=NC, num_subcores=NS)


def _edge_kernel(ei, hp):
    @functools.partial(
        pl.kernel,
        out_type=jax.ShapeDtypeStruct((NC * NP, D), jnp.float32),
        mesh=_sc_mesh(),
        scratch_types=[
            pltpu.VMEM((C, D), jnp.float32),
            pltpu.VMEM((C, D), jnp.float32),
            pltpu.VMEM((2, C), jnp.int32),
            pltpu.VMEM((2, C), jnp.int32),
            pltpu.VMEM((2, C), jnp.int32),
            pltpu.VMEM((2, C), jnp.int32),
            pltpu.VMEM_SHARED((NP, D), jnp.float32),
            pltpu.SemaphoreType.DMA,
            pltpu.SemaphoreType.DMA,
            pltpu.SemaphoreType.DMA,
            pltpu.SemaphoreType.DMA,
            pltpu.SemaphoreType.DMA,
            pltpu.SemaphoreType.DMA,
            pltpu.SemaphoreType.DMA,
            pltpu.SemaphoreType.DMA,
        ],
    )
    def k(ei_hbm, hp_hbm, part_hbm, rows0, rows1, ib0, ib1, ib2, ib3,
          accum, isem0, isem1, isem2, isem3, gsem0, gsem1,
          ssem0, ssem1):
        cidx = lax.axis_index("c")
        sidx = lax.axis_index("s")
        base = (cidx * NS + sidx) * EW

        rbase = sidx * PS
        last_s = sidx == NS - 1

        def init_from_hp(nrows):
            for kk in range(nrows // ZR):
                pltpu.sync_copy(hp_hbm.at[pl.ds(rbase + kk * ZR, ZR)], rows0)
                pltpu.sync_copy(rows0, accum.at[pl.ds(rbase + kk * ZR, ZR)])
            rem = nrows - (nrows // ZR) * ZR
            if rem:
                pltpu.sync_copy(
                    hp_hbm.at[pl.ds(rbase + (nrows // ZR) * ZR, rem)],
                    rows0.at[pl.ds(0, rem)])
                pltpu.sync_copy(
                    rows0.at[pl.ds(0, rem)],
                    accum.at[pl.ds(rbase + (nrows // ZR) * ZR, rem)])

        @pl.when(jnp.logical_and(cidx == 0, jnp.logical_not(last_s)))
        def _():
            init_from_hp(PS)

        @pl.when(jnp.logical_and(cidx == 0, last_s))
        def _():
            init_from_hp(N - (NS - 1) * PS)

        @pl.when(cidx != 0)
        def _():
            @pl.loop(0, C)
            def _(r):
                for q in range(D // 16):
                    rows0[r, pl.ds(q * 16, 16)] = jnp.zeros((16,), jnp.float32)
            for kk in range(PS // ZR):
                pltpu.sync_copy(rows0, accum.at[pl.ds(rbase + kk * ZR, ZR)])
            rem = PS - (PS // ZR) * ZR
            pltpu.sync_copy(rows0.at[pl.ds(0, rem)],
                            accum.at[pl.ds(rbase + (PS // ZR) * ZR, rem)])

        plsc.subcore_barrier()

        def load_idx(j, ib, sem):
            pltpu.async_copy(ei_hbm.at[:, pl.ds(base + j * C, C)], ib, sem)

        def idx_wait(ib, sem):
            pltpu.make_async_copy(ei_hbm.at[:, pl.ds(0, C)], ib, sem).wait()

        def gath(ib, rows, sem):
            pltpu.async_copy(hp_hbm.at[ib.at[0]], rows, sem)

        def gath_wait(ib, rows, sem):
            pltpu.make_async_copy(hp_hbm.at[ib.at[0]], rows, sem).wait()

        def scat(rows, ib, sem):
            pltpu.async_copy(rows, accum.at[ib.at[1]], sem, add=True)

        def scat_wait(rows, ib, sem):
            pltpu.make_async_copy(rows, accum.at[ib.at[1]], sem).wait()

        rbufs = ((rows0, gsem0, ssem0), (rows1, gsem1, ssem1))
        ibufs = ((ib0, isem0), (ib1, isem1), (ib2, isem2), (ib3, isem3))

        for b, (ib, isem) in enumerate(ibufs):
            load_idx(b, ib, isem)
        for b in range(2):
            rows, gsem, ssem = rbufs[b]
            ib, isem = ibufs[b]
            idx_wait(ib, isem)
            gath(ib, rows, gsem)

        def step(j, b4, load_ok=True, gath_ok=True):
            rows, gsem, ssem = rbufs[b4 % 2]
            ib, isem = ibufs[b4]
            gath_wait(ib, rows, gsem)
            scat(rows, ib, ssem)
            scat_wait(rows, ib, ssem)
            if load_ok:
                load_idx(j + 4, ib, isem)
            if gath_ok:
                nib, nisem = ibufs[(b4 + 2) % 4]
                idx_wait(nib, nisem)
                gath(nib, rows, gsem)

        @pl.loop(0, NCH // 4 - 1)
        def _(i):
            for b in range(4):
                step(i * 4 + b, b)

        for jj in range(NCH - 4, NCH):
            step(jj, jj % 4, load_ok=False, gath_ok=jj + 2 < NCH)

        plsc.subcore_barrier()
        pltpu.sync_copy(accum.at[pl.ds(rbase, PS)],
                        part_hbm.at[pl.ds(cidx * NP + rbase, PS)])

    return k(ei, hp)


def _deg_kernel(ei):
    @functools.partial(
        pl.kernel,
        out_type=jax.ShapeDtypeStruct((NC * NP, D), jnp.float32),
        mesh=_sc_mesh(),
        scratch_types=[
            pltpu.VMEM((C, D), jnp.float32),
            pltpu.VMEM((C, D), jnp.float32),
            pltpu.VMEM((2, C), jnp.int32),
            pltpu.VMEM((2, C), jnp.int32),
            pltpu.VMEM((2, C), jnp.int32),
            pltpu.VMEM_SHARED((NP, D), jnp.float32),
            pltpu.SemaphoreType.DMA,
            pltpu.SemaphoreType.DMA,
            pltpu.SemaphoreType.DMA,
            pltpu.SemaphoreType.DMA,
            pltpu.SemaphoreType.DMA,
            pltpu.SemaphoreType.DMA,
        ],
    )
    def k(ei_hbm, deg_hbm, ones_rows, zero_rows, ib0, ib1, ib2,
          accum, isem0, isem1, isem2, ssem0, ssem1, ssem2):
        cidx = lax.axis_index("c")
        sidx = lax.axis_index("s")
        base = (cidx * NS + sidx) * EW
        rbase = sidx * PS

        @pl.loop(0, C)
        def _(r):
            for q in range(D // 16):
                ones_rows[r, pl.ds(q * 16, 16)] = jnp.ones((16,), jnp.float32)
                zero_rows[r, pl.ds(q * 16, 16)] = jnp.zeros((16,), jnp.float32)

        @pl.when(cidx == 0)
        def _():
            for kk in range(PS // ZR):
                pltpu.sync_copy(ones_rows, accum.at[pl.ds(rbase + kk * ZR, ZR)])
            rem = PS - (PS // ZR) * ZR
            pltpu.sync_copy(ones_rows.at[pl.ds(0, rem)],
                            accum.at[pl.ds(rbase + (PS // ZR) * ZR, rem)])

        @pl.when(cidx != 0)
        def _():
            for kk in range(PS // ZR):
                pltpu.sync_copy(zero_rows, accum.at[pl.ds(rbase + kk * ZR, ZR)])
            rem = PS - (PS // ZR) * ZR
            pltpu.sync_copy(zero_rows.at[pl.ds(0, rem)],
                            accum.at[pl.ds(rbase + (PS // ZR) * ZR, rem)])

        plsc.subcore_barrier()

        bufs = ((ib0, isem0, ssem0), (ib1, isem1, ssem1), (ib2, isem2, ssem2))

        def load_idx(j, ib, sem):
            pltpu.async_copy(ei_hbm.at[:, pl.ds(base + j * C, C)], ib, sem)

        def idx_wait(ib, sem):
            pltpu.make_async_copy(ei_hbm.at[:, pl.ds(0, C)], ib, sem).wait()

        def scat(ib, sem):
            pltpu.async_copy(ones_rows, accum.at[ib.at[1]], sem, add=True)

        def scat_wait(ib, sem):
            pltpu.make_async_copy(ones_rows, accum.at[ib.at[1]], sem).wait()

        for b, (ib, isem, ssem) in enumerate(bufs):
            load_idx(b, ib, isem)

        LI = NCH // 3 - 1

        @pl.loop(0, LI)
        def _(i):
            j0 = i * 3
            for b, (ib, isem, ssem) in enumerate(bufs):
                idx_wait(ib, isem)
                scat(ib, ssem)
                scat_wait(ib, ssem)
                load_idx(j0 + b + 3, ib, isem)

        for b, (ib, isem, ssem) in enumerate(bufs):
            idx_wait(ib, isem)
            scat(ib, ssem)
            scat_wait(ib, ssem)
            if b < 2:
                load_idx(NCH - 2 + b, ib, isem)
        for b, (ib, isem, ssem) in enumerate(bufs[:2]):
            idx_wait(ib, isem)
            scat(ib, ssem)
            scat_wait(ib, ssem)

        plsc.subcore_barrier()
        pltpu.sync_copy(accum.at[pl.ds(rbase, PS)],
                        deg_hbm.at[pl.ds(cidx * NP + rbase, PS)])

    return k(ei)


def _tc_first(x, W, dinv_b):
    def body(x_ref, w_ref, d_ref, o_ref):
        h = jnp.dot(x_ref[...], w_ref[...], preferred_element_type=jnp.float32)
        o_ref[...] = h * lax.rsqrt(d_ref[:N, 0:1] + d_ref[NP:NP + N, 0:1])

    return pl.pallas_call(
        body, out_shape=jax.ShapeDtypeStruct((N, D), jnp.float32))(x, W, dinv_b)


def _tc_mid(part, dinv_b, b, W):
    def body(p_ref, d_ref, b_ref, w_ref, o_ref):
        agg = p_ref[:N, :] + p_ref[NP:NP + N, :]
        d = lax.rsqrt(d_ref[:N, 0:1] + d_ref[NP:NP + N, 0:1])
        a = jnp.maximum(agg * d + b_ref[...], 0.0)
        o_ref[...] = jnp.dot(
            a, w_ref[...], preferred_element_type=jnp.float32) * d

    return pl.pallas_call(
        body, out_shape=jax.ShapeDtypeStruct((N, D), jnp.float32))(
            part, dinv_b, b, W)


def _tc_final(part, dinv_b, b, bidx):
    def body(p_ref, d_ref, b_ref, bi_ref, o_ref):
        agg = p_ref[:N, :] + p_ref[NP:NP + N, :]
        d = lax.rsqrt(d_ref[:N, 0:1] + d_ref[NP:NP + N, 0:1])
        a = jnp.maximum(agg * d + b_ref[...], 0.0)
        gids = lax.broadcasted_iota(jnp.int32, (G, N), 0).astype(jnp.float32)
        mask = jnp.where(gids == bi_ref[0:1, :], 1.0, 0.0)
        sums = jnp.dot(mask, a, preferred_element_type=jnp.float32)
        counts = jnp.sum(mask, axis=1, keepdims=True)
        o_ref[...] = sums / jnp.maximum(counts, 1.0)

    return pl.pallas_call(
        body, out_shape=jax.ShapeDtypeStruct((G, D), jnp.float32))(
            part, dinv_b, b, bidx)


def kernel(x, edge_index, batch_idx, W1, b1, W2, b2, W3, b3):
    ei = edge_index.astype(jnp.int32)
    pad = jnp.broadcast_to(
        jnp.array([[0], [N]], jnp.int32), (2, EP - E))
    ei = jnp.concatenate([ei, pad], axis=1)
    bidx = jnp.broadcast_to(batch_idx.astype(jnp.float32)[None, :], (8, N))

    deg_b = _deg_kernel(ei)
    hp = _tc_first(x, W1, deg_b)
    part = _edge_kernel(ei, hp)
    hp = _tc_mid(part, deg_b, b1.reshape(1, D), W2)
    part = _edge_kernel(ei, hp)
    hp = _tc_mid(part, deg_b, b2.reshape(1, D), W3)
    part = _edge_kernel(ei, hp)
    return _tc_final(part, deg_b, b3.reshape(1, D), bidx)
